# Initial kernel scaffold; baseline (speedup 1.0000x reference)
import jax
import jax.numpy as jnp
from jax.experimental import pallas as pl


def kernel(x, adj_t, lin0_W, lin0_b, lin1_W, lin1_b, conv1_W1, conv1_W2, conv2_W1, conv2_W2):
    raise NotImplementedError("write your pallas kernel here")


# speedup vs baseline: 2.4703x; 2.4703x over previous
import functools
import math

import jax
import jax.numpy as jnp
from jax import lax
from jax.experimental import pallas as pl
from jax.experimental.pallas import tpu as pltpu
from jax.experimental.pallas import tpu_sc as plsc

N = 10000
E = 320000
F_IN = 128
H = 128
OUT = 64
ALPHA = 0.1
THETA = 0.5

NC = 2
NS = 16
NT = NC * NS
G = 128
GPT = 80
EPAD = NT * G * GPT
NPAD = 10240

RB = 2000


def _spmm_sc(h, srcg, dstg):
    mesh = plsc.VectorSubcoreMesh(core_axis_name="c", subcore_axis_name="s")

    @functools.partial(
        pl.kernel,
        out_type=jax.ShapeDtypeStruct((NC, N, H), jnp.float32),
        mesh=mesh,
        scratch_types=[
            pltpu.VMEM((GPT, G), jnp.int32),
            pltpu.VMEM((GPT, G), jnp.int32),
            pltpu.VMEM((G, H), jnp.float32),
            pltpu.VMEM_SHARED((NPAD, H), jnp.float32),
            pltpu.SemaphoreType.DMA,
        ],
    )
    def k(h_hbm, srcg_hbm, dstg_hbm, out_hbm, src_v, dst_v, rows_v, acc_sh, sem):
        c = lax.axis_index("c")
        s = lax.axis_index("s")
        wid = c * NS + s

        z = jnp.zeros((16,), jnp.float32)

        def zero_body(t, _):
            i = t // (H // 16)
            j = t % (H // 16)
            rows_v[i, pl.ds(j * 16, 16)] = z
            return 0

        lax.fori_loop(0, G * (H // 16), zero_body, 0)

        rows_per_tile = NPAD // NS

        def zcopy_body(kk, _):
            pltpu.sync_copy(rows_v, acc_sh.at[pl.ds(s * rows_per_tile + kk * G, G)])
            return 0

        lax.fori_loop(0, rows_per_tile // G, zcopy_body, 0)

        pltpu.sync_copy(srcg_hbm.at[pl.ds(wid * GPT, GPT)], src_v)
        pltpu.sync_copy(dstg_hbm.at[pl.ds(wid * GPT, GPT)], dst_v)

        plsc.subcore_barrier()

        def edge_body(g, _):
            pltpu.async_copy(h_hbm.at[src_v.at[g]], rows_v, sem).wait()
            pltpu.sync_copy(rows_v, acc_sh.at[dst_v.at[g]], add=True)
            return 0

        lax.fori_loop(0, GPT, edge_body, 0)

        plsc.subcore_barrier()

        out_rows = 624
        pltpu.sync_copy(acc_sh.at[pl.ds(s * out_rows, out_rows)],
                        out_hbm.at[c, pl.ds(s * out_rows, out_rows)])

        @pl.when(s == NS - 1)
        def _tail():
            pltpu.sync_copy(acc_sh.at[pl.ds(NS * out_rows, N - NS * out_rows)],
                            out_hbm.at[c, pl.ds(NS * out_rows, N - NS * out_rows)])

    return k(h, srcg, dstg)


def _dense0(x, w0t, b0, m2_1, m2_2):
    def body(x_ref, w_ref, b_ref, m1_ref, m2_ref, h_ref, b1_ref, b2_ref):
        h = jnp.maximum(
            jnp.dot(x_ref[...], w_ref[...], preferred_element_type=jnp.float32,
                    precision=lax.Precision.HIGHEST)
            + b_ref[...], 0.0)
        h_ref[...] = h
        b1_ref[...] = jnp.dot(h, m1_ref[...], preferred_element_type=jnp.float32,
                    precision=lax.Precision.HIGHEST)
        b2_ref[...] = jnp.dot(h, m2_ref[...], preferred_element_type=jnp.float32,
                    precision=lax.Precision.HIGHEST)

    grid = (N // RB,)
    o = jax.ShapeDtypeStruct((N, H), jnp.float32)
    return pl.pallas_call(
        body,
        grid=grid,
        in_specs=[
            pl.BlockSpec((RB, F_IN), lambda i: (i, 0)),
            pl.BlockSpec((F_IN, H), lambda i: (0, 0)),
            pl.BlockSpec((1, H), lambda i: (0, 0)),
            pl.BlockSpec((H, H), lambda i: (0, 0)),
            pl.BlockSpec((H, H), lambda i: (0, 0)),
        ],
        out_specs=[
            pl.BlockSpec((RB, H), lambda i: (i, 0)),
            pl.BlockSpec((RB, H), lambda i: (i, 0)),
            pl.BlockSpec((RB, H), lambda i: (i, 0)),
        ],
        out_shape=[o, o, o],
    )(x, w0t, b0, m2_1, m2_2)


def _combine(p, m1, b):
    def body(p_ref, m_ref, b_ref, o_ref):
        agg = p_ref[0] + p_ref[1]
        o_ref[...] = jnp.maximum(
            jnp.dot(agg, m_ref[...], preferred_element_type=jnp.float32,
                    precision=lax.Precision.HIGHEST)
            + b_ref[...], 0.0)

    return pl.pallas_call(
        body,
        grid=(N // RB,),
        in_specs=[
            pl.BlockSpec((NC, RB, H), lambda i: (0, i, 0)),
            pl.BlockSpec((H, H), lambda i: (0, 0)),
            pl.BlockSpec((RB, H), lambda i: (i, 0)),
        ],
        out_specs=pl.BlockSpec((RB, H), lambda i: (i, 0)),
        out_shape=jax.ShapeDtypeStruct((N, H), jnp.float32),
    )(p, m1, b)


def _final(p, m1, b, w1t, b1):
    def body(p_ref, m_ref, b_ref, w_ref, bias_ref, o_ref):
        agg = p_ref[0] + p_ref[1]
        h2 = jnp.maximum(
            jnp.dot(agg, m_ref[...], preferred_element_type=jnp.float32,
                    precision=lax.Precision.HIGHEST)
            + b_ref[...], 0.0)
        o_ref[...] = (jnp.dot(h2, w_ref[...], preferred_element_type=jnp.float32,
                    precision=lax.Precision.HIGHEST)
                      + bias_ref[...])

    return pl.pallas_call(
        body,
        grid=(N // RB,),
        in_specs=[
            pl.BlockSpec((NC, RB, H), lambda i: (0, i, 0)),
            pl.BlockSpec((H, H), lambda i: (0, 0)),
            pl.BlockSpec((RB, H), lambda i: (i, 0)),
            pl.BlockSpec((H, OUT), lambda i: (0, 0)),
            pl.BlockSpec((1, OUT), lambda i: (0, 0)),
        ],
        out_specs=pl.BlockSpec((RB, OUT), lambda i: (i, 0)),
        out_shape=jax.ShapeDtypeStruct((N, OUT), jnp.float32),
    )(p, m1, b, w1t, b1)


def kernel(x, adj_t, lin0_W, lin0_b, lin1_W, lin1_b,
           conv1_W1, conv1_W2, conv2_W1, conv2_W2):
    beta1 = float(math.log(THETA / 1 + 1.0))
    beta2 = float(math.log(THETA / 2 + 1.0))
    eye = jnp.eye(H, dtype=jnp.float32)
    m1_1 = (1.0 - ALPHA) * ((1.0 - beta1) * eye + beta1 * conv1_W1)
    m2_1 = ALPHA * ((1.0 - beta1) * eye + beta1 * conv1_W2)
    m1_2 = (1.0 - ALPHA) * ((1.0 - beta2) * eye + beta2 * conv2_W1)
    m2_2 = ALPHA * ((1.0 - beta2) * eye + beta2 * conv2_W2)

    pad = EPAD - E
    srcg = jnp.concatenate(
        [adj_t[0], jnp.zeros((pad,), jnp.int32)]).reshape(NT * GPT, G)
    dstg = jnp.concatenate(
        [adj_t[1], jnp.full((pad,), N, jnp.int32)]).reshape(NT * GPT, G)

    h, b1, b2 = _dense0(x, lin0_W.T, lin0_b.reshape(1, H), m2_1, m2_2)
    p1 = _spmm_sc(h, srcg, dstg)
    h1 = _combine(p1, m1_1, b1)
    p2 = _spmm_sc(h1, srcg, dstg)
    return _final(p2, m1_2, b2, lin1_W.T, lin1_b.reshape(1, OUT))


# speedup vs baseline: 3.2816x; 1.3284x over previous
import functools
import math

import jax
import jax.numpy as jnp
from jax import lax
from jax.experimental import pallas as pl
from jax.experimental.pallas import tpu as pltpu
from jax.experimental.pallas import tpu_sc as plsc

N = 10000
E = 320000
F_IN = 128
H = 128
OUT = 64
ALPHA = 0.1
THETA = 0.5

NC = 2
NS = 16
NT = NC * NS
G = 64
GPT = 160
EPAD = NT * G * GPT
NPAD = 10240
NBUF = 2

RB = 2000


def _spmm_sc(h, srcg, dstg):
    mesh = plsc.VectorSubcoreMesh(core_axis_name="c", subcore_axis_name="s")

    @functools.partial(
        pl.kernel,
        out_type=jax.ShapeDtypeStruct((NC, N, H), jnp.float32),
        mesh=mesh,
        scratch_types=[
            pltpu.VMEM((GPT // 2, 2 * G), jnp.int32),
            pltpu.VMEM((GPT, G), jnp.int32),
        ] + [pltpu.VMEM((G, H), jnp.float32)] * NBUF + [
            pltpu.VMEM_SHARED((NPAD, H), jnp.float32),
        ] + [pltpu.SemaphoreType.DMA] * NBUF,
    )
    def k(h_hbm, srcg_hbm, dstg_hbm, out_hbm, src_v, dst_v, *rest):
        rows = rest[:NBUF]
        acc_sh = rest[NBUF]
        sem_g = rest[NBUF + 1:NBUF + 1 + NBUF]
        c = lax.axis_index("c")
        s = lax.axis_index("s")
        wid = c * NS + s

        z = jnp.zeros((16,), jnp.float32)

        def zero_body(t, _):
            i = t // (H // 16)
            j = t % (H // 16)
            rows[0][i, pl.ds(j * 16, 16)] = z
            return 0

        lax.fori_loop(0, G * (H // 16), zero_body, 0)

        rows_per_tile = NPAD // NS

        def zcopy_body(kk, _):
            pltpu.sync_copy(rows[0], acc_sh.at[pl.ds(s * rows_per_tile + kk * G, G)])
            return 0

        lax.fori_loop(0, rows_per_tile // G, zcopy_body, 0)

        pltpu.sync_copy(srcg_hbm.at[pl.ds(wid * (GPT // 2), GPT // 2)], src_v)
        pltpu.sync_copy(dstg_hbm.at[pl.ds(wid * GPT, GPT)], dst_v)

        plsc.subcore_barrier()

        def src_idx(gg):
            half = lax.rem(gg, 2) * G
            return src_v.at[lax.div(gg, 2), pl.ds(pl.multiple_of(half, G), G)]

        for b in range(NBUF):
            pltpu.async_copy(h_hbm.at[src_idx(b)], rows[b], sem_g[b])

        def outer_body(go, _):
            for b in range(NBUF):
                gg = go * NBUF + b
                pltpu.make_async_copy(h_hbm.at[src_idx(gg)], rows[b],
                                      sem_g[b]).wait()
                pltpu.sync_copy(rows[b], acc_sh.at[dst_v.at[gg]], add=True)

                @pl.when(gg + NBUF < GPT)
                def _next():
                    pltpu.async_copy(h_hbm.at[src_idx(gg + NBUF)], rows[b],
                                     sem_g[b])
            return 0

        lax.fori_loop(0, GPT // NBUF, outer_body, 0)

        plsc.subcore_barrier()

        out_rows = 624
        pltpu.sync_copy(acc_sh.at[pl.ds(s * out_rows, out_rows)],
                        out_hbm.at[c, pl.ds(s * out_rows, out_rows)])

        @pl.when(s == NS - 1)
        def _tail():
            pltpu.sync_copy(acc_sh.at[pl.ds(NS * out_rows, N - NS * out_rows)],
                            out_hbm.at[c, pl.ds(NS * out_rows, N - NS * out_rows)])

    return k(h, srcg, dstg)


def _dense0(x, w0t, b0, m2_1, m2_2):
    def body(x_ref, w_ref, b_ref, m1_ref, m2_ref, h_ref, b1_ref, b2_ref):
        h = jnp.maximum(
            jnp.dot(x_ref[...], w_ref[...], preferred_element_type=jnp.float32,
                    precision=lax.Precision.HIGHEST)
            + b_ref[...], 0.0)
        h_ref[...] = h
        b1_ref[...] = jnp.dot(h, m1_ref[...], preferred_element_type=jnp.float32,
                              precision=lax.Precision.HIGHEST)
        b2_ref[...] = jnp.dot(h, m2_ref[...], preferred_element_type=jnp.float32,
                              precision=lax.Precision.HIGHEST)

    o = jax.ShapeDtypeStruct((N, H), jnp.float32)
    return pl.pallas_call(
        body,
        grid=(N // RB,),
        in_specs=[
            pl.BlockSpec((RB, F_IN), lambda i: (i, 0)),
            pl.BlockSpec((F_IN, H), lambda i: (0, 0)),
            pl.BlockSpec((1, H), lambda i: (0, 0)),
            pl.BlockSpec((H, H), lambda i: (0, 0)),
            pl.BlockSpec((H, H), lambda i: (0, 0)),
        ],
        out_specs=[
            pl.BlockSpec((RB, H), lambda i: (i, 0)),
            pl.BlockSpec((RB, H), lambda i: (i, 0)),
            pl.BlockSpec((RB, H), lambda i: (i, 0)),
        ],
        out_shape=[o, o, o],
    )(x, w0t, b0, m2_1, m2_2)


def _combine(p, m1, b):
    def body(p_ref, m_ref, b_ref, o_ref):
        agg = p_ref[0] + p_ref[1]
        o_ref[...] = jnp.maximum(
            jnp.dot(agg, m_ref[...], preferred_element_type=jnp.float32,
                    precision=lax.Precision.HIGHEST)
            + b_ref[...], 0.0)

    return pl.pallas_call(
        body,
        grid=(N // RB,),
        in_specs=[
            pl.BlockSpec((NC, RB, H), lambda i: (0, i, 0)),
            pl.BlockSpec((H, H), lambda i: (0, 0)),
            pl.BlockSpec((RB, H), lambda i: (i, 0)),
        ],
        out_specs=pl.BlockSpec((RB, H), lambda i: (i, 0)),
        out_shape=jax.ShapeDtypeStruct((N, H), jnp.float32),
    )(p, m1, b)


def _final(p, m1, b, w1t, b1):
    def body(p_ref, m_ref, b_ref, w_ref, bias_ref, o_ref):
        agg = p_ref[0] + p_ref[1]
        h2 = jnp.maximum(
            jnp.dot(agg, m_ref[...], preferred_element_type=jnp.float32,
                    precision=lax.Precision.HIGHEST)
            + b_ref[...], 0.0)
        o_ref[...] = (jnp.dot(h2, w_ref[...], preferred_element_type=jnp.float32,
                              precision=lax.Precision.HIGHEST)
                      + bias_ref[...])

    return pl.pallas_call(
        body,
        grid=(N // RB,),
        in_specs=[
            pl.BlockSpec((NC, RB, H), lambda i: (0, i, 0)),
            pl.BlockSpec((H, H), lambda i: (0, 0)),
            pl.BlockSpec((RB, H), lambda i: (i, 0)),
            pl.BlockSpec((H, OUT), lambda i: (0, 0)),
            pl.BlockSpec((1, OUT), lambda i: (0, 0)),
        ],
        out_specs=pl.BlockSpec((RB, OUT), lambda i: (i, 0)),
        out_shape=jax.ShapeDtypeStruct((N, OUT), jnp.float32),
    )(p, m1, b, w1t, b1)


def kernel(x, adj_t, lin0_W, lin0_b, lin1_W, lin1_b,
           conv1_W1, conv1_W2, conv2_W1, conv2_W2):
    beta1 = float(math.log(THETA / 1 + 1.0))
    beta2 = float(math.log(THETA / 2 + 1.0))
    eye = jnp.eye(H, dtype=jnp.float32)
    m1_1 = (1.0 - ALPHA) * ((1.0 - beta1) * eye + beta1 * conv1_W1)
    m2_1 = ALPHA * ((1.0 - beta1) * eye + beta1 * conv1_W2)
    m1_2 = (1.0 - ALPHA) * ((1.0 - beta2) * eye + beta2 * conv2_W1)
    m2_2 = ALPHA * ((1.0 - beta2) * eye + beta2 * conv2_W2)

    pad = EPAD - E
    srcg = jnp.concatenate(
        [adj_t[0], jnp.zeros((pad,), jnp.int32)]).reshape(NT * GPT // 2, 2 * G)
    dstg = jnp.concatenate(
        [adj_t[1], jnp.full((pad,), N, jnp.int32)]).reshape(NT * GPT, G)

    h, b1, b2 = _dense0(x, lin0_W.T, lin0_b.reshape(1, H), m2_1, m2_2)
    p1 = _spmm_sc(h, srcg, dstg)
    h1 = _combine(p1, m1_1, b1)
    p2 = _spmm_sc(h1, srcg, dstg)
    return _final(p2, m1_2, b2, lin1_W.T, lin1_b.reshape(1, OUT))
